# baseline (speedup 1.0000x reference)
import functools

import jax
import jax.numpy as jnp
from jax import lax
from jax.experimental import pallas as pl
from jax.experimental.pallas import tpu as pltpu
from jax.experimental.pallas import tpu_sc as plsc

N = 10000
H = 128
E = 320000
NC, NS, LANES = 2, 16, 16
CH = 128
EPAD = 327680
NCHUNK = EPAD // CH
CPT_FULL = NCHUNK // NS
CPT_HALF = NCHUNK // (2 * NS)
ACC_ROWS = 10016
DUMP = N
RPT = 624
SPANS = (128, 128, 128, 128, 112)
SBLK = 16
GSPLIT = 4


CPW = E // 8


def _counts_body(idx_hbm, out_hbm, idx_v, cnt_v):
    w = lax.axis_index("c") * NS + lax.axis_index("s")

    def zero(i, _):
        cnt_v[pl.ds(i * LANES, LANES)] = jnp.zeros((LANES,), jnp.float32)
        return 0
    lax.fori_loop(0, N // LANES, zero, 0)
    pltpu.sync_copy(idx_hbm.at[pl.ds(w * CPW, CPW)], idx_v)
    ones = jnp.ones((LANES,), jnp.float32)

    def body(i, _):
        idx = idx_v[pl.ds(i * LANES, LANES)]
        plsc.addupdate_scatter(cnt_v, [idx], ones)
        return 0
    lax.fori_loop(0, CPW // LANES, body, 0)
    pltpu.sync_copy(cnt_v, out_hbm.at[pl.ds(w * N, N)])


@functools.cache
def _counts_call():
    mesh = plsc.VectorSubcoreMesh(core_axis_name="c", subcore_axis_name="s",
                                  num_cores=NC, num_subcores=NS)
    return pl.kernel(
        _counts_body,
        out_type=jax.ShapeDtypeStruct((NC * NS * N,), jnp.float32),
        mesh=mesh,
        compiler_params=pltpu.CompilerParams(needs_layout_passes=False),
        scratch_types=[
            pltpu.VMEM((CPW,), jnp.int32),
            pltpu.VMEM((N,), jnp.float32),
        ],
    )


def _span_offsets():
    off, spans = 0, []
    for n in SPANS:
        spans.append((off, n))
        off += n
    return spans


def _zero_rows(rows_v):
    def zero(i, _):
        for j in range(H // LANES):
            rows_v[i, pl.ds(j * LANES, LANES)] = jnp.zeros((LANES,),
                                                           jnp.float32)
        return 0
    lax.fori_loop(0, CH, zero, 0)


def _run_agg(table, srcf, dstf, out, cpt, chunk0, init_table,
             acc, sidx_v, didx_v, rows_a, rows_b, gsem, ssem):
    s = lax.axis_index("s")
    row0 = s * RPT

    def init_span(r, n):
        if init_table:
            pltpu.sync_copy(table.at[pl.ds(r, n)], acc.at[pl.ds(r, n)])
        else:
            pltpu.sync_copy(rows_a.at[pl.ds(0, n)], acc.at[pl.ds(r, n)])

    if not init_table:
        _zero_rows(rows_a)
    for off, n in _span_offsets():
        init_span(row0 + off, n)

    @pl.when(s == 0)
    def _():
        init_span(NS * RPT, N - NS * RPT)
    plsc.subcore_barrier()
    cbase0 = chunk0 + s * cpt

    def gather(j, buf):
        for h in range(GSPLIT):
            g = CH // GSPLIT
            pltpu.async_copy(
                table.at[sidx_v.at[pl.ds(j * CH + h * g, g)]],
                buf.at[pl.ds(h * g, g)], gsem)

    def gwait(j, buf):
        pltpu.make_async_copy(table.at[sidx_v.at[pl.ds(j * CH, CH)]],
                              buf, gsem).wait()

    def scatter(j, buf):
        return pltpu.async_copy(buf, acc.at[didx_v.at[j]], ssem, add=True)

    def blk(b, _):
        cbase = cbase0 + b * SBLK
        pltpu.sync_copy(srcf.at[pl.ds(cbase * CH, SBLK * CH)], sidx_v)
        pltpu.sync_copy(dstf.at[pl.ds(cbase, SBLK)], didx_v)
        gather(0, rows_a)

        def pair(p, _):
            j0 = 2 * p
            gwait(j0, rows_a)
            gather(j0 + 1, rows_b)
            scatter(j0, rows_a).wait()
            gwait(j0 + 1, rows_b)

            @pl.when(j0 + 2 < SBLK)
            def _():
                gather(j0 + 2, rows_a)
            scatter(j0 + 1, rows_b).wait()
            return 0
        lax.fori_loop(0, SBLK // 2, pair, 0)
        return 0
    lax.fori_loop(0, cpt // SBLK, blk, 0)
    plsc.subcore_barrier()

    def wb_span(r, n):
        pltpu.sync_copy(acc.at[pl.ds(r, n)], out.at[pl.ds(r, n)])

    for off, n in _span_offsets():
        wb_span(row0 + off, n)

    @pl.when(s == 0)
    def _():
        wb_span(NS * RPT, N - NS * RPT)
    plsc.subcore_barrier()


def _agg_body(pp_src, pp_dst, dd_src, dd_dst,
              dt_d_src, dt_d_dst, dt_p_src, dt_p_dst,
              t_s, t_dd, t_pp, t_dt, t_td,
              o_s0, o_s1, o_dd, o_dt, o_pp, o_td,
              acc, sidx_v, didx_v, rows_a, rows_b, gsem, ssem):
    c = lax.axis_index("c")
    scr = (acc, sidx_v, didx_v, rows_a, rows_b, gsem, ssem)

    @pl.when(c == 0)
    def _():
        _run_agg(t_dd, dd_src, dd_dst, o_dd, CPT_FULL, 0, True, *scr)
        _run_agg(t_dt, dt_p_src, dt_d_dst, o_dt, CPT_FULL, 0, False, *scr)
        _run_agg(t_s, pp_src, pp_dst, o_s0, CPT_HALF, 0, True, *scr)

    @pl.when(c == 1)
    def _():
        _run_agg(t_pp, pp_src, pp_dst, o_pp, CPT_FULL, 0, True, *scr)
        _run_agg(t_td, dt_d_src, dt_p_dst, o_td, CPT_FULL, 0, False, *scr)
        _run_agg(t_s, pp_src, pp_dst, o_s1, CPT_HALF, NCHUNK // 2, False, *scr)


@functools.cache
def _agg_call():
    mesh = plsc.VectorSubcoreMesh(core_axis_name="c", subcore_axis_name="s",
                                  num_cores=NC, num_subcores=NS)
    return pl.kernel(
        _agg_body,
        out_type=[jax.ShapeDtypeStruct((N, H), jnp.float32)] * 6,
        mesh=mesh,
        compiler_params=pltpu.CompilerParams(needs_layout_passes=False),
        scratch_types=[
            pltpu.VMEM_SHARED((ACC_ROWS, H), jnp.float32),
            pltpu.VMEM((SBLK * CH,), jnp.int32),
            pltpu.VMEM((SBLK, CH), jnp.int32),
            pltpu.VMEM((CH, H), jnp.float32),
            pltpu.VMEM((CH, H), jnp.float32),
            pltpu.SemaphoreType.DMA,
            pltpu.SemaphoreType.DMA,
        ],
    )


BLK = 1000
GRID = N // BLK


def _scales_body(cnt_ref, out_ref):
    c = cnt_ref[...]
    dis_pp = lax.rsqrt(jnp.sum(c[0:8], axis=0) + 1.0)
    dis_dd = lax.rsqrt(jnp.sum(c[8:16], axis=0) + 1.0)
    cnt_d = jnp.sum(c[16:24], axis=0)
    cnt_p = jnp.sum(c[24:32], axis=0)
    inv_d = 1.0 / jnp.maximum(cnt_d, 1.0)
    msk_d = (cnt_d > 0).astype(jnp.float32)
    inv_p = 1.0 / jnp.maximum(cnt_p, 1.0)
    msk_p = (cnt_p > 0).astype(jnp.float32)
    out_ref[...] = jnp.stack(
        [dis_pp, dis_dd, inv_d, msk_d, inv_p, msk_p], axis=0)


def _seq_init_body(seq_ref, w_ref, dis_ref, x_ref, y_ref):
    x = jax.nn.relu(jnp.dot(seq_ref[...], w_ref[...],
                            preferred_element_type=jnp.float32))
    x_ref[...] = x
    y_ref[...] = x * dis_ref[...]


def _rowscale_body(x_ref, s_ref, y_ref):
    y_ref[...] = x_ref[...] * s_ref[...]


def _ln(t, g, b):
    mu = jnp.mean(t, axis=-1, keepdims=True)
    var = jnp.mean((t - mu) ** 2, axis=-1, keepdims=True)
    y = (t - mu) / jnp.sqrt(var + 1e-5)
    if g is not None:
        y = y * g + b
    return y


def _seq_update_body(a0_ref, a1_ref, dis_ref, x_ref, gw_ref, gb_ref,
                     lw_ref, g_ref, b_ref, *out_refs, do_relu, want_y):
    agg = (a0_ref[...] + a1_ref[...]) * dis_ref[...]
    pre = (jnp.dot(agg, gw_ref[...], preferred_element_type=jnp.float32)
           + gb_ref[...]
           + jnp.dot(x_ref[...], lw_ref[...],
                     preferred_element_type=jnp.float32) + 1e-6)
    t = _ln(pre, g_ref[...], b_ref[...])
    if do_relu:
        t = jax.nn.relu(t)
    out_refs[0][...] = t
    if want_y:
        out_refs[1][...] = t * dis_ref[...]


def _pkg_update_body(ag_ref, dis_ref, am_ref, inv_ref, msk_ref, x_ref,
                     gw_ref, gb_ref, mw_ref, mb_ref, xw_ref, xb_ref,
                     *out_refs, do_relu, want_y):
    dot = functools.partial(jnp.dot, preferred_element_type=jnp.float32)
    pre = (dot(ag_ref[...] * dis_ref[...], gw_ref[...]) + gb_ref[...]
           + dot(am_ref[...] * inv_ref[...], mw_ref[...])
           + msk_ref[...] * mb_ref[...]
           + dot(x_ref[...], xw_ref[...]) + xb_ref[...] + 1e-6)
    t = _ln(pre, None, None)
    if do_relu:
        t = jax.nn.relu(t)
    out_refs[0][...] = t
    if want_y:
        out_refs[1][...] = t * dis_ref[...]


def _rows(i):
    return (i, 0)


_B_NODE = pl.BlockSpec((BLK, H), _rows)
_B_COL = pl.BlockSpec((BLK, 1), _rows)
_B_W = pl.BlockSpec((H, H), lambda i: (0, 0))
_B_BIAS = pl.BlockSpec((1, H), lambda i: (0, 0))


def _node_out(n_out):
    return [jax.ShapeDtypeStruct((N, H), jnp.float32)] * n_out


_scales_call = pl.pallas_call(
    _scales_body,
    out_shape=jax.ShapeDtypeStruct((6, N), jnp.float32),
)

_seq_init_call = pl.pallas_call(
    _seq_init_body,
    grid=(GRID,),
    in_specs=[_B_NODE, _B_W, _B_COL],
    out_specs=[_B_NODE, _B_NODE],
    out_shape=_node_out(2),
)

_rowscale_call = pl.pallas_call(
    _rowscale_body,
    grid=(GRID,),
    in_specs=[_B_NODE, _B_COL],
    out_specs=_B_NODE,
    out_shape=jax.ShapeDtypeStruct((N, H), jnp.float32),
)


def _seq_update_call(do_relu, want_y):
    return pl.pallas_call(
        functools.partial(_seq_update_body, do_relu=do_relu, want_y=want_y),
        grid=(GRID,),
        in_specs=[_B_NODE, _B_NODE, _B_COL, _B_NODE, _B_W, _B_BIAS,
                  _B_W, _B_BIAS, _B_BIAS],
        out_specs=[_B_NODE] * (2 if want_y else 1),
        out_shape=_node_out(2 if want_y else 1),
    )


def _pkg_update_call(do_relu, want_y):
    return pl.pallas_call(
        functools.partial(_pkg_update_body, do_relu=do_relu, want_y=want_y),
        grid=(GRID,),
        in_specs=[_B_NODE, _B_COL, _B_NODE, _B_COL, _B_COL, _B_NODE,
                  _B_W, _B_BIAS, _B_W, _B_BIAS, _B_W, _B_BIAS],
        out_specs=[_B_NODE] * (2 if want_y else 1),
        out_shape=_node_out(2 if want_y else 1),
    )


def _prep_src(idx):
    return jnp.pad(idx, (0, EPAD - E), constant_values=0)


def _prep_dst(idx):
    return jnp.pad(idx, (0, EPAD - E),
                   constant_values=DUMP).reshape(NCHUNK, CH)


def kernel(seq, params, ppi, ddi, dti):
    def sorted_planes(src, dst):
        s, d = jax.lax.sort_key_val(src, dst)
        return _prep_src(s), _prep_dst(d)

    pp_src, pp_dst = sorted_planes(ppi[0], ppi[1])
    dd_src, dd_dst = sorted_planes(ddi[0], ddi[1])
    dt_p_src, dt_d_dst = sorted_planes(dti[1], dti[0])
    dt_d_src, dt_p_dst = sorted_planes(dti[0], dti[1])

    cnt_in = jnp.concatenate([ppi[1], ddi[1], dti[0], dti[1]])
    counts = _counts_call()(cnt_in)
    scl = _scales_call(counts.reshape(NC * NS, N))
    dis_pp = scl[0].reshape(N, 1)
    dis_dd = scl[1].reshape(N, 1)
    inv_d = scl[2].reshape(N, 1)
    msk_d = scl[3].reshape(N, 1)
    inv_p = scl[4].reshape(N, 1)
    msk_p = scl[5].reshape(N, 1)

    w0 = params['seq_init_W']
    x_seq, y_seq = _seq_init_call(seq, w0, dis_pp)

    xd0 = params['drug_emb']
    xp0 = params['prot_emb']
    y_d0 = _rowscale_call(xd0, dis_dd)
    y_p0 = _rowscale_call(xp0, dis_pp)

    def agg(t_s, t_dd, t_pp, t_dt, t_td):
        return _agg_call()(pp_src, pp_dst, dd_src, dd_dst,
                         dt_d_src, dt_d_dst, dt_p_src, dt_p_dst,
                         t_s, t_dd, t_pp, t_dt, t_td)

    o_s0, o_s1, o_dd, o_dt, o_pp, o_td = agg(y_seq, y_d0, y_p0, xp0, xd0)

    p1 = params['seq_l1']
    xl1, y_l2 = _seq_update_call(True, True)(
        o_s0, o_s1, dis_pp, x_seq, p1['gcn_W'], p1['gcn_b'].reshape(1, H),
        p1['lin_W'], p1['ln_g'].reshape(1, H), p1['ln_b'].reshape(1, H))

    k1 = params['pkg_l1']
    xd1, y_d1 = _pkg_update_call(True, True)(
        o_dd, dis_dd, o_dt, inv_d, msk_d, xd0,
        k1['gcn_dd_W'], k1['gcn_dd_b'].reshape(1, H),
        k1['lin_dt_W'], k1['lin_dt_b'].reshape(1, H),
        k1['lin_dr_W'], k1['lin_dr_b'].reshape(1, H))
    xp1, y_p1 = _pkg_update_call(True, True)(
        o_pp, dis_pp, o_td, inv_p, msk_p, xp0,
        k1['gcn_pp_W'], k1['gcn_pp_b'].reshape(1, H),
        k1['lin_td_W'], k1['lin_td_b'].reshape(1, H),
        k1['lin_pr_W'], k1['lin_pr_b'].reshape(1, H))

    s0, s1, a_dd, a_dt, a_pp, a_td = agg(y_l2, y_d1, y_p1, xp1, xd1)

    p2 = params['seq_l2']
    (x1,) = _seq_update_call(False, False)(
        s0, s1, dis_pp, xl1, p2['gcn_W'], p2['gcn_b'].reshape(1, H),
        p2['lin_W'], p2['ln_g'].reshape(1, H), p2['ln_b'].reshape(1, H))

    k2 = params['pkg_l2']
    (xd2,) = _pkg_update_call(False, False)(
        a_dd, dis_dd, a_dt, inv_d, msk_d, xd1,
        k2['gcn_dd_W'], k2['gcn_dd_b'].reshape(1, H),
        k2['lin_dt_W'], k2['lin_dt_b'].reshape(1, H),
        k2['lin_dr_W'], k2['lin_dr_b'].reshape(1, H))
    (xp2,) = _pkg_update_call(False, False)(
        a_pp, dis_pp, a_td, inv_p, msk_p, xp1,
        k2['gcn_pp_W'], k2['gcn_pp_b'].reshape(1, H),
        k2['lin_td_W'], k2['lin_td_b'].reshape(1, H),
        k2['lin_pr_W'], k2['lin_pr_b'].reshape(1, H))

    return (x_seq, xp0, x1, xp2, xd2)


# speedup vs baseline: 1.1906x; 1.1906x over previous
import functools

import jax
import jax.numpy as jnp
from jax import lax
from jax.experimental import pallas as pl
from jax.experimental.pallas import tpu as pltpu
from jax.experimental.pallas import tpu_sc as plsc

N = 10000
H = 128
E = 320000
NC, NS, LANES = 2, 16, 16
CH = 128
EPAD = 327680
NCHUNK = EPAD // CH
CPT_FULL = NCHUNK // NS
CPT_HALF = NCHUNK // (2 * NS)
ACC_ROWS = 10016
DUMP = N
RPT = 624
SPANS = (128, 128, 128, 128, 112)
SBLK = 16
GSPLIT = 4


CPW = E // 8


def _counts_body(idx_hbm, out_hbm, idx_v, cnt_v):
    w = lax.axis_index("c") * NS + lax.axis_index("s")

    def zero(i, _):
        cnt_v[pl.ds(i * LANES, LANES)] = jnp.zeros((LANES,), jnp.float32)
        return 0
    lax.fori_loop(0, N // LANES, zero, 0)
    pltpu.sync_copy(idx_hbm.at[pl.ds(w * CPW, CPW)], idx_v)
    ones = jnp.ones((LANES,), jnp.float32)

    def body(i, _):
        idx = idx_v[pl.ds(i * LANES, LANES)]
        plsc.addupdate_scatter(cnt_v, [idx], ones)
        return 0
    lax.fori_loop(0, CPW // LANES, body, 0)
    pltpu.sync_copy(cnt_v, out_hbm.at[pl.ds(w * N, N)])


@functools.cache
def _counts_call():
    mesh = plsc.VectorSubcoreMesh(core_axis_name="c", subcore_axis_name="s",
                                  num_cores=NC, num_subcores=NS)
    return pl.kernel(
        _counts_body,
        out_type=jax.ShapeDtypeStruct((NC * NS * N,), jnp.float32),
        mesh=mesh,
        compiler_params=pltpu.CompilerParams(needs_layout_passes=False),
        scratch_types=[
            pltpu.VMEM((CPW,), jnp.int32),
            pltpu.VMEM((N,), jnp.float32),
        ],
    )


def _span_offsets():
    off, spans = 0, []
    for n in SPANS:
        spans.append((off, n))
        off += n
    return spans


def _zero_rows(rows_v):
    def zero(i, _):
        for j in range(H // LANES):
            rows_v[i, pl.ds(j * LANES, LANES)] = jnp.zeros((LANES,),
                                                           jnp.float32)
        return 0
    lax.fori_loop(0, CH, zero, 0)


def _run_agg(table, srcf, dstf, out, cpt, chunk0, init_table,
             acc, sidx_v, didx_v, rows_a, rows_b, gsem, ssem):
    s = lax.axis_index("s")
    row0 = s * RPT

    def init_span(r, n):
        if init_table:
            pltpu.sync_copy(table.at[pl.ds(r, n)], acc.at[pl.ds(r, n)])
        else:
            pltpu.sync_copy(rows_a.at[pl.ds(0, n)], acc.at[pl.ds(r, n)])

    if not init_table:
        _zero_rows(rows_a)
    for off, n in _span_offsets():
        init_span(row0 + off, n)

    @pl.when(s == 0)
    def _():
        init_span(NS * RPT, N - NS * RPT)
    plsc.subcore_barrier()
    cbase0 = chunk0 + s * cpt

    def gather(j, buf):
        for h in range(GSPLIT):
            g = CH // GSPLIT
            pltpu.async_copy(
                table.at[sidx_v.at[pl.ds(j * CH + h * g, g)]],
                buf.at[pl.ds(h * g, g)], gsem)

    def gwait(j, buf):
        pltpu.make_async_copy(table.at[sidx_v.at[pl.ds(j * CH, CH)]],
                              buf, gsem).wait()

    def scatter(j, buf):
        return pltpu.async_copy(buf, acc.at[didx_v.at[j]], ssem, add=True)

    def blk(b, _):
        cbase = cbase0 + b * SBLK
        pltpu.sync_copy(srcf.at[pl.ds(cbase * CH, SBLK * CH)], sidx_v)
        pltpu.sync_copy(dstf.at[pl.ds(cbase, SBLK)], didx_v)
        gather(0, rows_a)

        def pair(p, _):
            j0 = 2 * p
            gwait(j0, rows_a)
            gather(j0 + 1, rows_b)
            scatter(j0, rows_a).wait()
            gwait(j0 + 1, rows_b)

            @pl.when(j0 + 2 < SBLK)
            def _():
                gather(j0 + 2, rows_a)
            scatter(j0 + 1, rows_b).wait()
            return 0
        lax.fori_loop(0, SBLK // 2, pair, 0)
        return 0
    lax.fori_loop(0, cpt // SBLK, blk, 0)
    plsc.subcore_barrier()

    def wb_span(r, n):
        pltpu.sync_copy(acc.at[pl.ds(r, n)], out.at[pl.ds(r, n)])

    for off, n in _span_offsets():
        wb_span(row0 + off, n)

    @pl.when(s == 0)
    def _():
        wb_span(NS * RPT, N - NS * RPT)
    plsc.subcore_barrier()


def _agg_body(pp_src, pp_dst, dd_src, dd_dst,
              dt_d_src, dt_d_dst, dt_p_src, dt_p_dst,
              t_s, t_dd, t_pp, t_dt, t_td,
              o_s0, o_s1, o_dd, o_dt, o_pp, o_td,
              acc, sidx_v, didx_v, rows_a, rows_b, gsem, ssem):
    c = lax.axis_index("c")
    scr = (acc, sidx_v, didx_v, rows_a, rows_b, gsem, ssem)

    @pl.when(c == 0)
    def _():
        _run_agg(t_dd, dd_src, dd_dst, o_dd, CPT_FULL, 0, True, *scr)
        _run_agg(t_dt, dt_p_src, dt_d_dst, o_dt, CPT_FULL, 0, False, *scr)
        _run_agg(t_s, pp_src, pp_dst, o_s0, CPT_HALF, 0, True, *scr)

    @pl.when(c == 1)
    def _():
        _run_agg(t_pp, pp_src, pp_dst, o_pp, CPT_FULL, 0, True, *scr)
        _run_agg(t_td, dt_d_src, dt_p_dst, o_td, CPT_FULL, 0, False, *scr)
        _run_agg(t_s, pp_src, pp_dst, o_s1, CPT_HALF, NCHUNK // 2, False, *scr)


@functools.cache
def _agg_call():
    mesh = plsc.VectorSubcoreMesh(core_axis_name="c", subcore_axis_name="s",
                                  num_cores=NC, num_subcores=NS)
    return pl.kernel(
        _agg_body,
        out_type=[jax.ShapeDtypeStruct((N, H), jnp.float32)] * 6,
        mesh=mesh,
        compiler_params=pltpu.CompilerParams(needs_layout_passes=False),
        scratch_types=[
            pltpu.VMEM_SHARED((ACC_ROWS, H), jnp.float32),
            pltpu.VMEM((SBLK * CH,), jnp.int32),
            pltpu.VMEM((SBLK, CH), jnp.int32),
            pltpu.VMEM((CH, H), jnp.float32),
            pltpu.VMEM((CH, H), jnp.float32),
            pltpu.SemaphoreType.DMA,
            pltpu.SemaphoreType.DMA,
        ],
    )


BLK = 1000
GRID = N // BLK


def _scales_body(cnt_ref, out_ref):
    c = cnt_ref[...]
    dis_pp = lax.rsqrt(jnp.sum(c[0:8], axis=0) + 1.0)
    dis_dd = lax.rsqrt(jnp.sum(c[8:16], axis=0) + 1.0)
    cnt_d = jnp.sum(c[16:24], axis=0)
    cnt_p = jnp.sum(c[24:32], axis=0)
    inv_d = 1.0 / jnp.maximum(cnt_d, 1.0)
    msk_d = (cnt_d > 0).astype(jnp.float32)
    inv_p = 1.0 / jnp.maximum(cnt_p, 1.0)
    msk_p = (cnt_p > 0).astype(jnp.float32)
    out_ref[...] = jnp.stack(
        [dis_pp, dis_dd, inv_d, msk_d, inv_p, msk_p], axis=0)


def _seq_init_body(seq_ref, w_ref, dis_ref, x_ref, y_ref):
    x = jax.nn.relu(jnp.dot(seq_ref[...], w_ref[...],
                            preferred_element_type=jnp.float32))
    x_ref[...] = x
    y_ref[...] = x * dis_ref[...]


def _rowscale_body(x_ref, s_ref, y_ref):
    y_ref[...] = x_ref[...] * s_ref[...]


def _ln(t, g, b):
    mu = jnp.mean(t, axis=-1, keepdims=True)
    var = jnp.mean((t - mu) ** 2, axis=-1, keepdims=True)
    y = (t - mu) / jnp.sqrt(var + 1e-5)
    if g is not None:
        y = y * g + b
    return y


def _seq_update_body(a0_ref, a1_ref, dis_ref, x_ref, gw_ref, gb_ref,
                     lw_ref, g_ref, b_ref, *out_refs, do_relu, want_y):
    agg = (a0_ref[...] + a1_ref[...]) * dis_ref[...]
    pre = (jnp.dot(agg, gw_ref[...], preferred_element_type=jnp.float32)
           + gb_ref[...]
           + jnp.dot(x_ref[...], lw_ref[...],
                     preferred_element_type=jnp.float32) + 1e-6)
    t = _ln(pre, g_ref[...], b_ref[...])
    if do_relu:
        t = jax.nn.relu(t)
    out_refs[0][...] = t
    if want_y:
        out_refs[1][...] = t * dis_ref[...]


def _pkg_update_body(ag_ref, dis_ref, am_ref, inv_ref, msk_ref, x_ref,
                     gw_ref, gb_ref, mw_ref, mb_ref, xw_ref, xb_ref,
                     *out_refs, do_relu, want_y):
    dot = functools.partial(jnp.dot, preferred_element_type=jnp.float32)
    pre = (dot(ag_ref[...] * dis_ref[...], gw_ref[...]) + gb_ref[...]
           + dot(am_ref[...] * inv_ref[...], mw_ref[...])
           + msk_ref[...] * mb_ref[...]
           + dot(x_ref[...], xw_ref[...]) + xb_ref[...] + 1e-6)
    t = _ln(pre, None, None)
    if do_relu:
        t = jax.nn.relu(t)
    out_refs[0][...] = t
    if want_y:
        out_refs[1][...] = t * dis_ref[...]


def _rows(i):
    return (i, 0)


_B_NODE = pl.BlockSpec((BLK, H), _rows)
_B_COL = pl.BlockSpec((BLK, 1), _rows)
_B_W = pl.BlockSpec((H, H), lambda i: (0, 0))
_B_BIAS = pl.BlockSpec((1, H), lambda i: (0, 0))


def _node_out(n_out):
    return [jax.ShapeDtypeStruct((N, H), jnp.float32)] * n_out


_scales_call = pl.pallas_call(
    _scales_body,
    out_shape=jax.ShapeDtypeStruct((6, N), jnp.float32),
)

_seq_init_call = pl.pallas_call(
    _seq_init_body,
    grid=(GRID,),
    in_specs=[_B_NODE, _B_W, _B_COL],
    out_specs=[_B_NODE, _B_NODE],
    out_shape=_node_out(2),
)

_rowscale_call = pl.pallas_call(
    _rowscale_body,
    grid=(GRID,),
    in_specs=[_B_NODE, _B_COL],
    out_specs=_B_NODE,
    out_shape=jax.ShapeDtypeStruct((N, H), jnp.float32),
)


def _seq_update_call(do_relu, want_y):
    return pl.pallas_call(
        functools.partial(_seq_update_body, do_relu=do_relu, want_y=want_y),
        grid=(GRID,),
        in_specs=[_B_NODE, _B_NODE, _B_COL, _B_NODE, _B_W, _B_BIAS,
                  _B_W, _B_BIAS, _B_BIAS],
        out_specs=[_B_NODE] * (2 if want_y else 1),
        out_shape=_node_out(2 if want_y else 1),
    )


def _pkg_update_call(do_relu, want_y):
    return pl.pallas_call(
        functools.partial(_pkg_update_body, do_relu=do_relu, want_y=want_y),
        grid=(GRID,),
        in_specs=[_B_NODE, _B_COL, _B_NODE, _B_COL, _B_COL, _B_NODE,
                  _B_W, _B_BIAS, _B_W, _B_BIAS, _B_W, _B_BIAS],
        out_specs=[_B_NODE] * (2 if want_y else 1),
        out_shape=_node_out(2 if want_y else 1),
    )


def _prep_src(idx):
    return jnp.pad(idx, (0, EPAD - E), constant_values=0)


def _prep_dst(idx):
    return jnp.pad(idx, (0, EPAD - E),
                   constant_values=DUMP).reshape(NCHUNK, CH)


def kernel(seq, params, ppi, ddi, dti):
    def sorted_planes(src, dst):
        key = lax.sort(src * 16384 + dst, is_stable=False)
        return _prep_src(key >> 14), _prep_dst(key & 16383)

    pp_src, pp_dst = sorted_planes(ppi[0], ppi[1])
    dd_src, dd_dst = sorted_planes(ddi[0], ddi[1])
    dt_p_src, dt_d_dst = sorted_planes(dti[1], dti[0])
    dt_d_src, dt_p_dst = sorted_planes(dti[0], dti[1])

    cnt_in = jnp.concatenate([ppi[1], ddi[1], dti[0], dti[1]])
    counts = _counts_call()(cnt_in)
    scl = _scales_call(counts.reshape(NC * NS, N))
    dis_pp = scl[0].reshape(N, 1)
    dis_dd = scl[1].reshape(N, 1)
    inv_d = scl[2].reshape(N, 1)
    msk_d = scl[3].reshape(N, 1)
    inv_p = scl[4].reshape(N, 1)
    msk_p = scl[5].reshape(N, 1)

    w0 = params['seq_init_W']
    x_seq, y_seq = _seq_init_call(seq, w0, dis_pp)

    xd0 = params['drug_emb']
    xp0 = params['prot_emb']
    y_d0 = _rowscale_call(xd0, dis_dd)
    y_p0 = _rowscale_call(xp0, dis_pp)

    def agg(t_s, t_dd, t_pp, t_dt, t_td):
        return _agg_call()(pp_src, pp_dst, dd_src, dd_dst,
                         dt_d_src, dt_d_dst, dt_p_src, dt_p_dst,
                         t_s, t_dd, t_pp, t_dt, t_td)

    o_s0, o_s1, o_dd, o_dt, o_pp, o_td = agg(y_seq, y_d0, y_p0, xp0, xd0)

    p1 = params['seq_l1']
    xl1, y_l2 = _seq_update_call(True, True)(
        o_s0, o_s1, dis_pp, x_seq, p1['gcn_W'], p1['gcn_b'].reshape(1, H),
        p1['lin_W'], p1['ln_g'].reshape(1, H), p1['ln_b'].reshape(1, H))

    k1 = params['pkg_l1']
    xd1, y_d1 = _pkg_update_call(True, True)(
        o_dd, dis_dd, o_dt, inv_d, msk_d, xd0,
        k1['gcn_dd_W'], k1['gcn_dd_b'].reshape(1, H),
        k1['lin_dt_W'], k1['lin_dt_b'].reshape(1, H),
        k1['lin_dr_W'], k1['lin_dr_b'].reshape(1, H))
    xp1, y_p1 = _pkg_update_call(True, True)(
        o_pp, dis_pp, o_td, inv_p, msk_p, xp0,
        k1['gcn_pp_W'], k1['gcn_pp_b'].reshape(1, H),
        k1['lin_td_W'], k1['lin_td_b'].reshape(1, H),
        k1['lin_pr_W'], k1['lin_pr_b'].reshape(1, H))

    s0, s1, a_dd, a_dt, a_pp, a_td = agg(y_l2, y_d1, y_p1, xp1, xd1)

    p2 = params['seq_l2']
    (x1,) = _seq_update_call(False, False)(
        s0, s1, dis_pp, xl1, p2['gcn_W'], p2['gcn_b'].reshape(1, H),
        p2['lin_W'], p2['ln_g'].reshape(1, H), p2['ln_b'].reshape(1, H))

    k2 = params['pkg_l2']
    (xd2,) = _pkg_update_call(False, False)(
        a_dd, dis_dd, a_dt, inv_d, msk_d, xd1,
        k2['gcn_dd_W'], k2['gcn_dd_b'].reshape(1, H),
        k2['lin_dt_W'], k2['lin_dt_b'].reshape(1, H),
        k2['lin_dr_W'], k2['lin_dr_b'].reshape(1, H))
    (xp2,) = _pkg_update_call(False, False)(
        a_pp, dis_pp, a_td, inv_p, msk_p, xp1,
        k2['gcn_pp_W'], k2['gcn_pp_b'].reshape(1, H),
        k2['lin_td_W'], k2['lin_td_b'].reshape(1, H),
        k2['lin_pr_W'], k2['lin_pr_b'].reshape(1, H))

    return (x_seq, xp0, x1, xp2, xd2)


# speedup vs baseline: 1.7629x; 1.4807x over previous
import functools

import jax
import jax.numpy as jnp
from jax import lax
from jax.experimental import pallas as pl
from jax.experimental.pallas import tpu as pltpu
from jax.experimental.pallas import tpu_sc as plsc

N = 10000
H = 128
E = 320000
NC, NS, LANES = 2, 16, 16
CH = 128
EPAD = 327680
NCHUNK = EPAD // CH
CPT_FULL = NCHUNK // NS
CPT_HALF = NCHUNK // (2 * NS)
ACC_ROWS = 10016
DUMP = N
RPT = 624
SPANS = (128, 128, 128, 128, 112)
SBLK = 32


CPW = E // 8


def _counts_body(idx_hbm, out_hbm, idx_v, cnt_v):
    w = lax.axis_index("c") * NS + lax.axis_index("s")

    def zero(i, _):
        cnt_v[pl.ds(i * LANES, LANES)] = jnp.zeros((LANES,), jnp.float32)
        return 0
    lax.fori_loop(0, N // LANES, zero, 0)
    pltpu.sync_copy(idx_hbm.at[pl.ds(w * CPW, CPW)], idx_v)
    ones = jnp.ones((LANES,), jnp.float32)

    def body(i, _):
        idx = idx_v[pl.ds(i * LANES, LANES)]
        plsc.addupdate_scatter(cnt_v, [idx], ones)
        return 0
    lax.fori_loop(0, CPW // LANES, body, 0)
    pltpu.sync_copy(cnt_v, out_hbm.at[pl.ds(w * N, N)])


@functools.cache
def _counts_call():
    mesh = plsc.VectorSubcoreMesh(core_axis_name="c", subcore_axis_name="s",
                                  num_cores=NC, num_subcores=NS)
    return pl.kernel(
        _counts_body,
        out_type=jax.ShapeDtypeStruct((NC * NS * N,), jnp.float32),
        mesh=mesh,
        compiler_params=pltpu.CompilerParams(needs_layout_passes=False),
        scratch_types=[
            pltpu.VMEM((CPW,), jnp.int32),
            pltpu.VMEM((N,), jnp.float32),
        ],
    )


def _span_offsets():
    off, spans = 0, []
    for n in SPANS:
        spans.append((off, n))
        off += n
    return spans


def _zero_rows(rows_v):
    def zero(i, _):
        for j in range(H // LANES):
            rows_v[i, pl.ds(j * LANES, LANES)] = jnp.zeros((LANES,),
                                                           jnp.float32)
        return 0
    lax.fori_loop(0, CH, zero, 0)


def _run_agg(table, srcf, dstf, out, cpt, chunk0, init_table,
             acc, sidx_v, didx_v, rows_a, rows_b, gsem, ssem):
    s = lax.axis_index("s")
    row0 = s * RPT

    def init_span(r, n):
        if init_table:
            pltpu.sync_copy(table.at[pl.ds(r, n)], acc.at[pl.ds(r, n)])
        else:
            pltpu.sync_copy(rows_a.at[pl.ds(0, n)], acc.at[pl.ds(r, n)])

    if not init_table:
        _zero_rows(rows_a)
    for off, n in _span_offsets():
        init_span(row0 + off, n)

    @pl.when(s == 0)
    def _():
        init_span(NS * RPT, N - NS * RPT)
    plsc.subcore_barrier()
    cbase0 = chunk0 + s * cpt

    def gather(j, buf):
        return pltpu.async_copy(table.at[sidx_v.at[pl.ds(j * CH, CH)]],
                                buf, gsem)

    def scatter(j, buf):
        return pltpu.async_copy(buf, acc.at[didx_v.at[j]], ssem, add=True)

    def blk(b, _):
        cbase = cbase0 + b * SBLK
        pltpu.sync_copy(srcf.at[pl.ds(cbase * CH, SBLK * CH)], sidx_v)
        pltpu.sync_copy(dstf.at[pl.ds(cbase, SBLK)], didx_v)
        gather(0, rows_a)

        def pair(p, _):
            j0 = 2 * p
            pltpu.make_async_copy(table.at[sidx_v.at[pl.ds(j0 * CH, CH)]],
                                  rows_a, gsem).wait()
            gb = gather(j0 + 1, rows_b)
            scatter(j0, rows_a).wait()
            gb.wait()

            @pl.when(j0 + 2 < SBLK)
            def _():
                gather(j0 + 2, rows_a)
            scatter(j0 + 1, rows_b).wait()
            return 0
        lax.fori_loop(0, SBLK // 2, pair, 0)
        return 0
    lax.fori_loop(0, cpt // SBLK, blk, 0)
    plsc.subcore_barrier()

    def wb_span(r, n):
        pltpu.sync_copy(acc.at[pl.ds(r, n)], out.at[pl.ds(r, n)])

    for off, n in _span_offsets():
        wb_span(row0 + off, n)

    @pl.when(s == 0)
    def _():
        wb_span(NS * RPT, N - NS * RPT)
    plsc.subcore_barrier()


def _agg_body(pp_src, pp_dst, dd_src, dd_dst,
              dt_d_src, dt_d_dst, dt_p_src, dt_p_dst,
              t_s, t_dd, t_pp, t_dt, t_td,
              o_s0, o_s1, o_dd, o_dt, o_pp, o_td,
              acc, sidx_v, didx_v, rows_a, rows_b, gsem, ssem):
    c = lax.axis_index("c")
    scr = (acc, sidx_v, didx_v, rows_a, rows_b, gsem, ssem)

    @pl.when(c == 0)
    def _():
        _run_agg(t_dd, dd_src, dd_dst, o_dd, CPT_FULL, 0, True, *scr)
        _run_agg(t_dt, dt_p_src, dt_d_dst, o_dt, CPT_FULL, 0, False, *scr)
        _run_agg(t_s, pp_src, pp_dst, o_s0, CPT_HALF, 0, True, *scr)

    @pl.when(c == 1)
    def _():
        _run_agg(t_pp, pp_src, pp_dst, o_pp, CPT_FULL, 0, True, *scr)
        _run_agg(t_td, dt_d_src, dt_p_dst, o_td, CPT_FULL, 0, False, *scr)
        _run_agg(t_s, pp_src, pp_dst, o_s1, CPT_HALF, NCHUNK // 2, False, *scr)


@functools.cache
def _agg_call():
    mesh = plsc.VectorSubcoreMesh(core_axis_name="c", subcore_axis_name="s",
                                  num_cores=NC, num_subcores=NS)
    return pl.kernel(
        _agg_body,
        out_type=[jax.ShapeDtypeStruct((N, H), jnp.float32)] * 6,
        mesh=mesh,
        compiler_params=pltpu.CompilerParams(needs_layout_passes=False),
        scratch_types=[
            pltpu.VMEM_SHARED((ACC_ROWS, H), jnp.float32),
            pltpu.VMEM((SBLK * CH,), jnp.int32),
            pltpu.VMEM((SBLK, CH), jnp.int32),
            pltpu.VMEM((CH, H), jnp.float32),
            pltpu.VMEM((CH, H), jnp.float32),
            pltpu.SemaphoreType.DMA,
            pltpu.SemaphoreType.DMA,
        ],
    )


BLK = 1000
GRID = N // BLK


def _scales_body(cnt_ref, out_ref):
    c = cnt_ref[...]
    dis_pp = lax.rsqrt(jnp.sum(c[0:8], axis=0) + 1.0)
    dis_dd = lax.rsqrt(jnp.sum(c[8:16], axis=0) + 1.0)
    cnt_d = jnp.sum(c[16:24], axis=0)
    cnt_p = jnp.sum(c[24:32], axis=0)
    inv_d = 1.0 / jnp.maximum(cnt_d, 1.0)
    msk_d = (cnt_d > 0).astype(jnp.float32)
    inv_p = 1.0 / jnp.maximum(cnt_p, 1.0)
    msk_p = (cnt_p > 0).astype(jnp.float32)
    out_ref[...] = jnp.stack(
        [dis_pp, dis_dd, inv_d, msk_d, inv_p, msk_p], axis=0)


def _seq_init_body(seq_ref, w_ref, dis_ref, x_ref, y_ref):
    x = jax.nn.relu(jnp.dot(seq_ref[...], w_ref[...],
                            preferred_element_type=jnp.float32))
    x_ref[...] = x
    y_ref[...] = x * dis_ref[...]


def _rowscale_body(x_ref, s_ref, y_ref):
    y_ref[...] = x_ref[...] * s_ref[...]


def _ln(t, g, b):
    mu = jnp.mean(t, axis=-1, keepdims=True)
    var = jnp.mean((t - mu) ** 2, axis=-1, keepdims=True)
    y = (t - mu) / jnp.sqrt(var + 1e-5)
    if g is not None:
        y = y * g + b
    return y


def _seq_update_body(a0_ref, a1_ref, dis_ref, x_ref, gw_ref, gb_ref,
                     lw_ref, g_ref, b_ref, *out_refs, do_relu, want_y):
    agg = (a0_ref[...] + a1_ref[...]) * dis_ref[...]
    pre = (jnp.dot(agg, gw_ref[...], preferred_element_type=jnp.float32)
           + gb_ref[...]
           + jnp.dot(x_ref[...], lw_ref[...],
                     preferred_element_type=jnp.float32) + 1e-6)
    t = _ln(pre, g_ref[...], b_ref[...])
    if do_relu:
        t = jax.nn.relu(t)
    out_refs[0][...] = t
    if want_y:
        out_refs[1][...] = t * dis_ref[...]


def _pkg_update_body(ag_ref, dis_ref, am_ref, inv_ref, msk_ref, x_ref,
                     gw_ref, gb_ref, mw_ref, mb_ref, xw_ref, xb_ref,
                     *out_refs, do_relu, want_y):
    dot = functools.partial(jnp.dot, preferred_element_type=jnp.float32)
    pre = (dot(ag_ref[...] * dis_ref[...], gw_ref[...]) + gb_ref[...]
           + dot(am_ref[...] * inv_ref[...], mw_ref[...])
           + msk_ref[...] * mb_ref[...]
           + dot(x_ref[...], xw_ref[...]) + xb_ref[...] + 1e-6)
    t = _ln(pre, None, None)
    if do_relu:
        t = jax.nn.relu(t)
    out_refs[0][...] = t
    if want_y:
        out_refs[1][...] = t * dis_ref[...]


def _rows(i):
    return (i, 0)


_B_NODE = pl.BlockSpec((BLK, H), _rows)
_B_COL = pl.BlockSpec((BLK, 1), _rows)
_B_W = pl.BlockSpec((H, H), lambda i: (0, 0))
_B_BIAS = pl.BlockSpec((1, H), lambda i: (0, 0))


def _node_out(n_out):
    return [jax.ShapeDtypeStruct((N, H), jnp.float32)] * n_out


_scales_call = pl.pallas_call(
    _scales_body,
    out_shape=jax.ShapeDtypeStruct((6, N), jnp.float32),
)

_seq_init_call = pl.pallas_call(
    _seq_init_body,
    grid=(GRID,),
    in_specs=[_B_NODE, _B_W, _B_COL],
    out_specs=[_B_NODE, _B_NODE],
    out_shape=_node_out(2),
)

_rowscale_call = pl.pallas_call(
    _rowscale_body,
    grid=(GRID,),
    in_specs=[_B_NODE, _B_COL],
    out_specs=_B_NODE,
    out_shape=jax.ShapeDtypeStruct((N, H), jnp.float32),
)


def _seq_update_call(do_relu, want_y):
    return pl.pallas_call(
        functools.partial(_seq_update_body, do_relu=do_relu, want_y=want_y),
        grid=(GRID,),
        in_specs=[_B_NODE, _B_NODE, _B_COL, _B_NODE, _B_W, _B_BIAS,
                  _B_W, _B_BIAS, _B_BIAS],
        out_specs=[_B_NODE] * (2 if want_y else 1),
        out_shape=_node_out(2 if want_y else 1),
    )


def _pkg_update_call(do_relu, want_y):
    return pl.pallas_call(
        functools.partial(_pkg_update_body, do_relu=do_relu, want_y=want_y),
        grid=(GRID,),
        in_specs=[_B_NODE, _B_COL, _B_NODE, _B_COL, _B_COL, _B_NODE,
                  _B_W, _B_BIAS, _B_W, _B_BIAS, _B_W, _B_BIAS],
        out_specs=[_B_NODE] * (2 if want_y else 1),
        out_shape=_node_out(2 if want_y else 1),
    )


def _prep_src(idx):
    return jnp.pad(idx, (0, EPAD - E), constant_values=0)


def _prep_dst(idx):
    return jnp.pad(idx, (0, EPAD - E),
                   constant_values=DUMP).reshape(NCHUNK, CH)


def kernel(seq, params, ppi, ddi, dti):
    pp_src = _prep_src(ppi[0])
    pp_dst = _prep_dst(ppi[1])
    dd_src = _prep_src(ddi[0])
    dd_dst = _prep_dst(ddi[1])
    dt_d_src = _prep_src(dti[0])
    dt_d_dst = _prep_dst(dti[0])
    dt_p_src = _prep_src(dti[1])
    dt_p_dst = _prep_dst(dti[1])

    cnt_in = jnp.concatenate([ppi[1], ddi[1], dti[0], dti[1]])
    counts = _counts_call()(cnt_in)
    scl = _scales_call(counts.reshape(NC * NS, N))
    dis_pp = scl[0].reshape(N, 1)
    dis_dd = scl[1].reshape(N, 1)
    inv_d = scl[2].reshape(N, 1)
    msk_d = scl[3].reshape(N, 1)
    inv_p = scl[4].reshape(N, 1)
    msk_p = scl[5].reshape(N, 1)

    w0 = params['seq_init_W']
    x_seq, y_seq = _seq_init_call(seq, w0, dis_pp)

    xd0 = params['drug_emb']
    xp0 = params['prot_emb']
    y_d0 = _rowscale_call(xd0, dis_dd)
    y_p0 = _rowscale_call(xp0, dis_pp)

    def agg(t_s, t_dd, t_pp, t_dt, t_td):
        return _agg_call()(pp_src, pp_dst, dd_src, dd_dst,
                         dt_d_src, dt_d_dst, dt_p_src, dt_p_dst,
                         t_s, t_dd, t_pp, t_dt, t_td)

    o_s0, o_s1, o_dd, o_dt, o_pp, o_td = agg(y_seq, y_d0, y_p0, xp0, xd0)

    p1 = params['seq_l1']
    xl1, y_l2 = _seq_update_call(True, True)(
        o_s0, o_s1, dis_pp, x_seq, p1['gcn_W'], p1['gcn_b'].reshape(1, H),
        p1['lin_W'], p1['ln_g'].reshape(1, H), p1['ln_b'].reshape(1, H))

    k1 = params['pkg_l1']
    xd1, y_d1 = _pkg_update_call(True, True)(
        o_dd, dis_dd, o_dt, inv_d, msk_d, xd0,
        k1['gcn_dd_W'], k1['gcn_dd_b'].reshape(1, H),
        k1['lin_dt_W'], k1['lin_dt_b'].reshape(1, H),
        k1['lin_dr_W'], k1['lin_dr_b'].reshape(1, H))
    xp1, y_p1 = _pkg_update_call(True, True)(
        o_pp, dis_pp, o_td, inv_p, msk_p, xp0,
        k1['gcn_pp_W'], k1['gcn_pp_b'].reshape(1, H),
        k1['lin_td_W'], k1['lin_td_b'].reshape(1, H),
        k1['lin_pr_W'], k1['lin_pr_b'].reshape(1, H))

    s0, s1, a_dd, a_dt, a_pp, a_td = agg(y_l2, y_d1, y_p1, xp1, xd1)

    p2 = params['seq_l2']
    (x1,) = _seq_update_call(False, False)(
        s0, s1, dis_pp, xl1, p2['gcn_W'], p2['gcn_b'].reshape(1, H),
        p2['lin_W'], p2['ln_g'].reshape(1, H), p2['ln_b'].reshape(1, H))

    k2 = params['pkg_l2']
    (xd2,) = _pkg_update_call(False, False)(
        a_dd, dis_dd, a_dt, inv_d, msk_d, xd1,
        k2['gcn_dd_W'], k2['gcn_dd_b'].reshape(1, H),
        k2['lin_dt_W'], k2['lin_dt_b'].reshape(1, H),
        k2['lin_dr_W'], k2['lin_dr_b'].reshape(1, H))
    (xp2,) = _pkg_update_call(False, False)(
        a_pp, dis_pp, a_td, inv_p, msk_p, xp1,
        k2['gcn_pp_W'], k2['gcn_pp_b'].reshape(1, H),
        k2['lin_td_W'], k2['lin_td_b'].reshape(1, H),
        k2['lin_pr_W'], k2['lin_pr_b'].reshape(1, H))

    return (x_seq, xp0, x1, xp2, xd2)
